# baseline (speedup 1.0000x reference)
import functools

import jax
import jax.numpy as jnp
from jax import lax
from jax.experimental import pallas as pl
from jax.experimental.pallas import tpu as pltpu
from jax.experimental.pallas import tpu_sc as plsc

N_NODES = 10000
N_EDGES = 320000
F_IN = 128
F_OUT = 64
F_HALF = F_OUT // 2

NC = 2
NS = 16
LANES = 16
NW = NC * NS

CH = 128
NROWS = N_EDGES // CH
ROWS_BASE = NROWS // NW
ROWS_EXTRA = NROWS % NW
ROWS_BASE16 = NROWS // NS
ROWS_EXTRA16 = NROWS % NS
N_PAD = 10240
DEG_PER_SUB = N_PAD // NS
ACC_PER_SUB = N_NODES // NS
NBUF = 4

_mesh = plsc.VectorSubcoreMesh(core_axis_name="c", subcore_axis_name="s")
_sc_params = pltpu.CompilerParams(use_tc_tiling_on_sc=False)


@functools.partial(
    pl.kernel,
    out_type=(
        jax.ShapeDtypeStruct((NC, N_PAD), jnp.float32),
        jax.ShapeDtypeStruct((NC, N_PAD), jnp.float32),
    ),
    mesh=_mesh,
    compiler_params=_sc_params,
    scratch_types=[
        pltpu.VMEM((ROWS_BASE + 1, CH), jnp.int32),
        pltpu.VMEM((ROWS_BASE + 1, CH), jnp.int32),
        pltpu.VMEM((CH,), jnp.float32),
        pltpu.VMEM_SHARED((N_PAD,), jnp.float32),
        pltpu.VMEM_SHARED((N_PAD,), jnp.float32),
        pltpu.SemaphoreType.DMA,
    ],
)
def _degrees_kernel(e3d_hbm, z1_hbm, odeg_hbm, ideg_hbm,
                    idxs_v, idxd_v, ones_v, odeg_sp, ideg_sp, sem):
    c = lax.axis_index("c")
    s = lax.axis_index("s")
    wid = s * NC + c
    base = ROWS_BASE * wid + jnp.minimum(wid, ROWS_EXTRA)
    nrows = ROWS_BASE + jnp.where(wid < ROWS_EXTRA, 1, 0)
    sl = pl.ds(s * DEG_PER_SUB, DEG_PER_SUB)
    pltpu.sync_copy(z1_hbm, odeg_sp.at[sl])
    pltpu.sync_copy(z1_hbm, ideg_sp.at[sl])
    for i in range(CH // LANES):
        ones_v[pl.ds(i * LANES, LANES)] = jnp.ones((LANES,), jnp.float32)
    pltpu.sync_copy(e3d_hbm.at[0].at[pl.ds(base, ROWS_BASE)],
                    idxs_v.at[pl.ds(0, ROWS_BASE)])
    pltpu.sync_copy(e3d_hbm.at[1].at[pl.ds(base, ROWS_BASE)],
                    idxd_v.at[pl.ds(0, ROWS_BASE)])

    @pl.when(wid < ROWS_EXTRA)
    def _():
        pltpu.sync_copy(e3d_hbm.at[0].at[pl.ds(base + ROWS_BASE, 1)],
                        idxs_v.at[pl.ds(ROWS_BASE, 1)])
        pltpu.sync_copy(e3d_hbm.at[1].at[pl.ds(base + ROWS_BASE, 1)],
                        idxd_v.at[pl.ds(ROWS_BASE, 1)])

    plsc.subcore_barrier()

    def body(g, carry):
        pltpu.async_copy(ones_v, odeg_sp.at[idxs_v.at[g]], sem, add=True)
        pltpu.async_copy(ones_v, ideg_sp.at[idxd_v.at[g]], sem, add=True)
        return carry

    lax.fori_loop(0, nrows, body, 0)

    def drain(g, carry):
        pltpu.make_async_copy(ones_v, odeg_sp.at[idxs_v.at[0]], sem).wait()
        pltpu.make_async_copy(ones_v, ideg_sp.at[idxd_v.at[0]], sem).wait()
        return carry

    lax.fori_loop(0, nrows, drain, 0)
    plsc.subcore_barrier()
    pltpu.sync_copy(odeg_sp.at[sl], odeg_hbm.at[c].at[sl])
    pltpu.sync_copy(ideg_sp.at[sl], ideg_hbm.at[c].at[sl])


BLK = 1024


def _matmul_body(x_ref, w_ref, y_ref):
    y_ref[...] = jnp.dot(x_ref[...], w_ref[...],
                         preferred_element_type=jnp.float32)


_matmul = pl.pallas_call(
    _matmul_body,
    grid=(pl.cdiv(N_NODES, BLK),),
    in_specs=[
        pl.BlockSpec((BLK, F_IN), lambda i: (i, 0)),
        pl.BlockSpec((F_IN, F_OUT), lambda i: (0, 0)),
    ],
    out_specs=pl.BlockSpec((BLK, F_OUT), lambda i: (i, 0)),
    out_shape=jax.ShapeDtypeStruct((N_NODES, F_OUT), jnp.float32),
)


def _scale_body(y_ref, odeg_ref, ideg_ref, y2_ref, nr_ref):
    norm_l = lax.rsqrt(jnp.maximum(odeg_ref[0, :] + odeg_ref[1, :], 1.0))
    y = y_ref[...] * norm_l[:, None]
    y2_ref[0] = y[:, :F_HALF]
    y2_ref[1] = y[:, F_HALF:]
    nr_ref[0, :] = lax.rsqrt(jnp.maximum(ideg_ref[0, :] + ideg_ref[1, :], 1.0))


_scale = pl.pallas_call(
    _scale_body,
    grid=(pl.cdiv(N_NODES, BLK),),
    in_specs=[
        pl.BlockSpec((BLK, F_OUT), lambda i: (i, 0)),
        pl.BlockSpec((2, BLK), lambda i: (0, i)),
        pl.BlockSpec((2, BLK), lambda i: (0, i)),
    ],
    out_specs=[
        pl.BlockSpec((2, BLK, F_HALF), lambda i: (0, i, 0)),
        pl.BlockSpec((1, BLK), lambda i: (0, i)),
    ],
    out_shape=[
        jax.ShapeDtypeStruct((2, N_NODES, F_HALF), jnp.float32),
        jax.ShapeDtypeStruct((1, N_PAD), jnp.float32),
    ],
)


@functools.partial(
    pl.kernel,
    out_type=jax.ShapeDtypeStruct((N_NODES, F_OUT), jnp.float32),
    mesh=_mesh,
    compiler_params=_sc_params,
    scratch_types=[
        pltpu.VMEM((ROWS_BASE16 + 1, CH), jnp.int32),
        pltpu.VMEM((ROWS_BASE16 + 1, CH), jnp.int32),
        [pltpu.VMEM((CH, F_HALF), jnp.float32)] * NBUF,
        pltpu.VMEM((ACC_PER_SUB, F_HALF), jnp.float32),
        pltpu.VMEM((DEG_PER_SUB + 8,), jnp.float32),
        pltpu.VMEM_SHARED((N_NODES, F_HALF), jnp.float32),
        [pltpu.SemaphoreType.DMA] * NBUF,
        [pltpu.SemaphoreType.DMA] * NBUF,
    ],
)
def _aggregate_kernel(y2_hbm, e3d_hbm, nr_hbm, zrow_hbm,
                      out_hbm,
                      idxs_v, idxd_v, rows, accv, normv,
                      acc_sp, sg, ss):
    c = lax.axis_index("c")
    s = lax.axis_index("s")
    base = ROWS_BASE16 * s + jnp.minimum(s, ROWS_EXTRA16)
    nrows = ROWS_BASE16 + jnp.where(s < ROWS_EXTRA16, 1, 0)
    sl_acc = pl.ds(s * ACC_PER_SUB, ACC_PER_SUB)
    y_hbm = y2_hbm.at[c]
    pltpu.sync_copy(zrow_hbm, acc_sp.at[sl_acc])
    pltpu.sync_copy(e3d_hbm.at[0].at[pl.ds(base, ROWS_BASE16)],
                    idxs_v.at[pl.ds(0, ROWS_BASE16)])
    pltpu.sync_copy(e3d_hbm.at[1].at[pl.ds(base, ROWS_BASE16)],
                    idxd_v.at[pl.ds(0, ROWS_BASE16)])

    @pl.when(s < ROWS_EXTRA16)
    def _():
        pltpu.sync_copy(e3d_hbm.at[0].at[pl.ds(base + ROWS_BASE16, 1)],
                        idxs_v.at[pl.ds(ROWS_BASE16, 1)])
        pltpu.sync_copy(e3d_hbm.at[1].at[pl.ds(base + ROWS_BASE16, 1)],
                        idxd_v.at[pl.ds(ROWS_BASE16, 1)])

    al = (s * ACC_PER_SUB) // 8 * 8
    delta = s * ACC_PER_SUB - al
    pltpu.sync_copy(nr_hbm.at[0].at[pl.ds(al, DEG_PER_SUB + 8)], normv)
    plsc.subcore_barrier()

    for b in range(NBUF):
        pltpu.async_copy(y_hbm.at[idxs_v.at[b]], rows[b], sg[b])

    def block(g, b):
        @pl.when(g < nrows)
        def _():
            pltpu.make_async_copy(y_hbm.at[idxs_v.at[g]], rows[b], sg[b]).wait()
            pltpu.async_copy(rows[b], acc_sp.at[idxd_v.at[g]], ss[b], add=True)
            j = g - 2
            jb = (b - 2) % NBUF

            @pl.when(jnp.logical_and(j >= 0, j + NBUF < nrows))
            def _():
                pltpu.make_async_copy(rows[jb], acc_sp.at[idxd_v.at[0]],
                                      ss[jb]).wait()
                pltpu.async_copy(y_hbm.at[idxs_v.at[j + NBUF]], rows[jb],
                                 sg[jb])

    def body(i, carry):
        for b in range(NBUF):
            block(i * NBUF + b, b)
        return carry

    lax.fori_loop(0, (ROWS_BASE16 + 1 + NBUF - 1) // NBUF, body, 0)

    for b in range(NBUF):
        pltpu.make_async_copy(rows[b], acc_sp.at[idxd_v.at[0]], ss[b]).wait()

    plsc.subcore_barrier()

    pltpu.sync_copy(acc_sp.at[sl_acc], accv)

    def scale_row(r, carry):
        v = normv[pl.ds(delta + r, LANES)]
        nv = jnp.broadcast_to(v[0], (LANES,))
        accv[r, pl.ds(0, LANES)] = accv[r, pl.ds(0, LANES)] * nv
        accv[r, pl.ds(LANES, LANES)] = accv[r, pl.ds(LANES, LANES)] * nv
        return carry

    lax.fori_loop(0, ACC_PER_SUB, scale_row, 0)
    pltpu.sync_copy(accv,
                    out_hbm.at[sl_acc, pl.ds(c * F_HALF, F_HALF)])


@jax.jit
def kernel(x, edge_index, W):
    e3d = jnp.reshape(edge_index.astype(jnp.int32), (2, NROWS, CH))
    z1 = jnp.zeros((DEG_PER_SUB,), jnp.float32)
    zrow = jnp.zeros((ACC_PER_SUB, F_HALF), jnp.float32)

    out_deg, in_deg = _degrees_kernel(e3d, z1)
    y_raw = _matmul(x, W)
    y2, norm_r = _scale(y_raw, out_deg, in_deg)
    return _aggregate_kernel(y2, e3d, norm_r, zrow)


# speedup vs baseline: 1.1133x; 1.1133x over previous
import functools

import jax
import jax.numpy as jnp
from jax import lax
from jax.experimental import pallas as pl
from jax.experimental.pallas import tpu as pltpu
from jax.experimental.pallas import tpu_sc as plsc

N_NODES = 10000
N_EDGES = 320000
F_IN = 128
F_OUT = 64

NC = 2
NS = 16
LANES = 16
NW = NC * NS

CH = 128
NROWS = N_EDGES // CH
ROWS_BASE = NROWS // NW
ROWS_EXTRA = NROWS % NW
N_PAD = 10240
DEG_PER_SUB = N_PAD // NS
ACC_PER_SUB = N_NODES // NS

_mesh = plsc.VectorSubcoreMesh(core_axis_name="c", subcore_axis_name="s")
_sc_params = pltpu.CompilerParams(use_tc_tiling_on_sc=False)


def _tile_rows(wid):
    base = ROWS_BASE * wid + jnp.minimum(wid, ROWS_EXTRA)
    n = ROWS_BASE + jnp.where(wid < ROWS_EXTRA, 1, 0)
    return base, n


def _load_tile_rows(src3d, base, wid, idx_v):
    pltpu.sync_copy(src3d.at[pl.ds(base, ROWS_BASE)],
                    idx_v.at[pl.ds(0, ROWS_BASE)])

    @pl.when(wid < ROWS_EXTRA)
    def _():
        pltpu.sync_copy(src3d.at[pl.ds(base + ROWS_BASE, 1)],
                        idx_v.at[pl.ds(ROWS_BASE, 1)])


@functools.partial(
    pl.kernel,
    out_type=jax.ShapeDtypeStruct((NC, N_PAD), jnp.float32),
    mesh=_mesh,
    compiler_params=_sc_params,
    scratch_types=[
        pltpu.VMEM((ROWS_BASE + 1, CH), jnp.int32),
        pltpu.VMEM((CH,), jnp.float32),
        pltpu.VMEM_SHARED((N_PAD,), jnp.float32),
        pltpu.SemaphoreType.DMA,
    ],
)
def _out_degree_kernel(e3d_hbm, z1_hbm, deg_hbm, idx_v, ones_v, deg_sp, sem):
    c = lax.axis_index("c")
    s = lax.axis_index("s")
    wid = s * NC + c
    base, nrows = _tile_rows(wid)
    sl = pl.ds(s * DEG_PER_SUB, DEG_PER_SUB)
    pltpu.sync_copy(z1_hbm, deg_sp.at[sl])
    for i in range(CH // LANES):
        ones_v[pl.ds(i * LANES, LANES)] = jnp.ones((LANES,), jnp.float32)
    _load_tile_rows(e3d_hbm.at[0], base, wid, idx_v)
    plsc.subcore_barrier()

    def body(g, carry):
        pltpu.async_copy(ones_v, deg_sp.at[idx_v.at[g]], sem, add=True)
        return carry

    lax.fori_loop(0, nrows, body, 0)

    def drain(g, carry):
        pltpu.make_async_copy(ones_v, deg_sp.at[idx_v.at[0]], sem).wait()
        return carry

    lax.fori_loop(0, nrows, drain, 0)
    plsc.subcore_barrier()
    pltpu.sync_copy(deg_sp.at[sl], deg_hbm.at[c].at[sl])


@functools.partial(
    pl.kernel,
    out_type=(
        jax.ShapeDtypeStruct((NC, N_NODES, F_OUT), jnp.float32),
        jax.ShapeDtypeStruct((NC, N_PAD), jnp.float32),
    ),
    mesh=_mesh,
    compiler_params=_sc_params,
    scratch_types=[
        pltpu.VMEM((ROWS_BASE + 1, CH), jnp.int32),
        pltpu.VMEM((ROWS_BASE + 1, CH), jnp.int32),
        [pltpu.VMEM((CH, F_OUT), jnp.float32)] * 8,
        pltpu.VMEM((CH,), jnp.float32),
        pltpu.VMEM_SHARED((N_NODES, F_OUT), jnp.float32),
        pltpu.VMEM_SHARED((N_PAD,), jnp.float32),
        [pltpu.SemaphoreType.DMA] * 8,
        [pltpu.SemaphoreType.DMA] * 8,
        pltpu.SemaphoreType.DMA,
    ],
)
def _aggregate_kernel(y_hbm, e3d_hbm, zrow_hbm, z1_hbm,
                      part_hbm, indeg_hbm,
                      idxs_v, idxd_v, rows, ones_v,
                      acc_sp, deg_sp, sg, ss, semd):
    c = lax.axis_index("c")
    s = lax.axis_index("s")
    wid = s * NC + c
    base, nrows = _tile_rows(wid)
    sl_acc = pl.ds(s * ACC_PER_SUB, ACC_PER_SUB)
    sl_deg = pl.ds(s * DEG_PER_SUB, DEG_PER_SUB)
    pltpu.sync_copy(zrow_hbm, acc_sp.at[sl_acc])
    pltpu.sync_copy(z1_hbm, deg_sp.at[sl_deg])
    for i in range(CH // LANES):
        ones_v[pl.ds(i * LANES, LANES)] = jnp.ones((LANES,), jnp.float32)
    _load_tile_rows(e3d_hbm.at[0], base, wid, idxs_v)
    _load_tile_rows(e3d_hbm.at[1], base, wid, idxd_v)
    plsc.subcore_barrier()

    NBUF = 8
    for b in range(NBUF):
        pltpu.async_copy(y_hbm.at[idxs_v.at[b]], rows[b], sg[b])

    def block(g, b):
        @pl.when(g < nrows)
        def _():
            pltpu.make_async_copy(y_hbm.at[idxs_v.at[g]], rows[b], sg[b]).wait()
            pltpu.async_copy(rows[b], acc_sp.at[idxd_v.at[g]], ss[b], add=True)
            pltpu.async_copy(ones_v, deg_sp.at[idxd_v.at[g]], semd, add=True)
            j = g - 4
            jb = (b - 4) % NBUF

            @pl.when(jnp.logical_and(j >= 0, j + NBUF < nrows))
            def _():
                pltpu.make_async_copy(rows[jb], acc_sp.at[idxd_v.at[0]],
                                      ss[jb]).wait()
                pltpu.async_copy(y_hbm.at[idxs_v.at[j + NBUF]], rows[jb],
                                 sg[jb])

    def body(i, carry):
        for b in range(NBUF):
            block(i * NBUF + b, b)
        return carry

    lax.fori_loop(0, (ROWS_BASE + 1 + NBUF - 1) // NBUF, body, 0)

    for b in range(NBUF):
        pltpu.make_async_copy(rows[b], acc_sp.at[idxd_v.at[0]], ss[b]).wait()

    def drain(g, carry):
        pltpu.make_async_copy(ones_v, deg_sp.at[idxd_v.at[0]], semd).wait()
        return carry

    lax.fori_loop(0, nrows, drain, 0)
    plsc.subcore_barrier()
    pltpu.sync_copy(acc_sp.at[sl_acc], part_hbm.at[c].at[sl_acc])
    pltpu.sync_copy(deg_sp.at[sl_deg], indeg_hbm.at[c].at[sl_deg])


BLK = 1024


def _matmul_body(deg_ref, x_ref, w_ref, y_ref):
    deg = deg_ref[0, :] + deg_ref[1, :]
    norm = lax.rsqrt(jnp.maximum(deg, 1.0))
    y_ref[...] = jnp.dot(x_ref[...] * norm[:, None], w_ref[...],
                         preferred_element_type=jnp.float32)


_matmul = pl.pallas_call(
    _matmul_body,
    grid=(pl.cdiv(N_NODES, BLK),),
    in_specs=[
        pl.BlockSpec((2, BLK), lambda i: (0, i)),
        pl.BlockSpec((BLK, F_IN), lambda i: (i, 0)),
        pl.BlockSpec((F_IN, F_OUT), lambda i: (0, 0)),
    ],
    out_specs=pl.BlockSpec((BLK, F_OUT), lambda i: (i, 0)),
    out_shape=jax.ShapeDtypeStruct((N_NODES, F_OUT), jnp.float32),
)


def _finalize_body(part_ref, indeg_ref, out_ref):
    acc = part_ref[0] + part_ref[1]
    deg = indeg_ref[0, :] + indeg_ref[1, :]
    norm = lax.rsqrt(jnp.maximum(deg, 1.0))
    out_ref[...] = acc * norm[:, None]


_finalize = pl.pallas_call(
    _finalize_body,
    grid=(pl.cdiv(N_NODES, BLK),),
    in_specs=[
        pl.BlockSpec((2, BLK, F_OUT), lambda i: (0, i, 0)),
        pl.BlockSpec((2, BLK), lambda i: (0, i)),
    ],
    out_specs=pl.BlockSpec((BLK, F_OUT), lambda i: (i, 0)),
    out_shape=jax.ShapeDtypeStruct((N_NODES, F_OUT), jnp.float32),
)


@jax.jit
def kernel(x, edge_index, W):
    e3d = jnp.reshape(edge_index.astype(jnp.int32), (2, NROWS, CH))
    z1 = jnp.zeros((DEG_PER_SUB,), jnp.float32)
    zrow = jnp.zeros((ACC_PER_SUB, F_OUT), jnp.float32)

    out_deg = _out_degree_kernel(e3d, z1)
    y = _matmul(out_deg, x, W)
    part, in_deg = _aggregate_kernel(y, e3d, zrow, z1)
    return _finalize(part, in_deg)


# speedup vs baseline: 1.1529x; 1.0355x over previous
import functools

import jax
import jax.numpy as jnp
from jax import lax
from jax.experimental import pallas as pl
from jax.experimental.pallas import tpu as pltpu
from jax.experimental.pallas import tpu_sc as plsc

N_NODES = 10000
N_EDGES = 320000
F_IN = 128
F_OUT = 64

NC = 2
NS = 16
LANES = 16
NW = NC * NS

CH = 128
NROWS = N_EDGES // CH
ROWS_BASE = NROWS // NW
ROWS_EXTRA = NROWS % NW
N_PAD = 10240
DEG_PER_SUB = N_PAD // NS
ACC_PER_SUB = N_NODES // NS

_mesh = plsc.VectorSubcoreMesh(core_axis_name="c", subcore_axis_name="s")
_sc_params = pltpu.CompilerParams(use_tc_tiling_on_sc=False)


def _tile_rows(wid):
    base = ROWS_BASE * wid + jnp.minimum(wid, ROWS_EXTRA)
    n = ROWS_BASE + jnp.where(wid < ROWS_EXTRA, 1, 0)
    return base, n


def _load_tile_rows(src3d, base, wid, idx_v):
    pltpu.sync_copy(src3d.at[pl.ds(base, ROWS_BASE)],
                    idx_v.at[pl.ds(0, ROWS_BASE)])

    @pl.when(wid < ROWS_EXTRA)
    def _():
        pltpu.sync_copy(src3d.at[pl.ds(base + ROWS_BASE, 1)],
                        idx_v.at[pl.ds(ROWS_BASE, 1)])


@functools.partial(
    pl.kernel,
    out_type=jax.ShapeDtypeStruct((NC, N_PAD), jnp.float32),
    mesh=_mesh,
    compiler_params=_sc_params,
    scratch_types=[
        pltpu.VMEM((ROWS_BASE + 1, CH), jnp.int32),
        pltpu.VMEM((CH,), jnp.float32),
        pltpu.VMEM_SHARED((N_PAD,), jnp.float32),
        pltpu.SemaphoreType.DMA,
    ],
)
def _out_degree_kernel(e3d_hbm, z1_hbm, deg_hbm, idx_v, ones_v, deg_sp, sem):
    c = lax.axis_index("c")
    s = lax.axis_index("s")
    wid = s * NC + c
    base, nrows = _tile_rows(wid)
    sl = pl.ds(s * DEG_PER_SUB, DEG_PER_SUB)
    pltpu.sync_copy(z1_hbm, deg_sp.at[sl])
    for i in range(CH // LANES):
        ones_v[pl.ds(i * LANES, LANES)] = jnp.ones((LANES,), jnp.float32)
    _load_tile_rows(e3d_hbm.at[0], base, wid, idx_v)
    plsc.subcore_barrier()

    def body(g, carry):
        pltpu.async_copy(ones_v, deg_sp.at[idx_v.at[g]], sem, add=True)
        return carry

    lax.fori_loop(0, nrows, body, 0)

    def drain(g, carry):
        pltpu.make_async_copy(ones_v, deg_sp.at[idx_v.at[0]], sem).wait()
        return carry

    lax.fori_loop(0, nrows, drain, 0)
    plsc.subcore_barrier()
    pltpu.sync_copy(deg_sp.at[sl], deg_hbm.at[c].at[sl])


@functools.partial(
    pl.kernel,
    out_type=(
        jax.ShapeDtypeStruct((NC, N_NODES, F_OUT), jnp.float32),
        jax.ShapeDtypeStruct((NC, N_PAD), jnp.float32),
    ),
    mesh=_mesh,
    compiler_params=_sc_params,
    scratch_types=[
        pltpu.VMEM((ROWS_BASE + 1, CH), jnp.int32),
        pltpu.VMEM((ROWS_BASE + 1, CH), jnp.int32),
        [pltpu.VMEM((CH, F_OUT), jnp.float32)] * 8,
        pltpu.VMEM((CH,), jnp.float32),
        pltpu.VMEM_SHARED((N_NODES, F_OUT), jnp.float32),
        pltpu.VMEM_SHARED((N_PAD,), jnp.float32),
        [pltpu.SemaphoreType.DMA] * 8,
        [pltpu.SemaphoreType.DMA] * 8,
        pltpu.SemaphoreType.DMA,
    ],
)
def _aggregate_kernel(y_hbm, e3d_hbm, zrow_hbm, z1_hbm,
                      part_hbm, indeg_hbm,
                      idxs_v, idxd_v, rows, ones_v,
                      acc_sp, deg_sp, sg, ss, semd):
    c = lax.axis_index("c")
    s = lax.axis_index("s")
    wid = s * NC + c
    base, nrows = _tile_rows(wid)
    sl_acc = pl.ds(s * ACC_PER_SUB, ACC_PER_SUB)
    sl_deg = pl.ds(s * DEG_PER_SUB, DEG_PER_SUB)
    pltpu.sync_copy(zrow_hbm, acc_sp.at[sl_acc])
    pltpu.sync_copy(z1_hbm, deg_sp.at[sl_deg])
    for i in range(CH // LANES):
        ones_v[pl.ds(i * LANES, LANES)] = jnp.ones((LANES,), jnp.float32)
    _load_tile_rows(e3d_hbm.at[0], base, wid, idxs_v)
    _load_tile_rows(e3d_hbm.at[1], base, wid, idxd_v)
    plsc.subcore_barrier()

    NBUF = 8
    for b in range(NBUF):
        pltpu.async_copy(y_hbm.at[idxs_v.at[b]], rows[b], sg[b])

    def block(g, b):
        @pl.when(g < nrows)
        def _():
            pltpu.make_async_copy(y_hbm.at[idxs_v.at[g]], rows[b], sg[b]).wait()
            pltpu.async_copy(rows[b], acc_sp.at[idxd_v.at[g]], ss[b], add=True)
            pltpu.async_copy(ones_v, deg_sp.at[idxd_v.at[g]], semd, add=True)
            j = g - 4
            jb = (b - 4) % NBUF

            @pl.when(jnp.logical_and(j >= 0, j + NBUF < nrows))
            def _():
                pltpu.make_async_copy(rows[jb], acc_sp.at[idxd_v.at[0]],
                                      ss[jb]).wait()
                pltpu.async_copy(y_hbm.at[idxs_v.at[j + NBUF]], rows[jb],
                                 sg[jb])

    def body(i, carry):
        for b in range(NBUF):
            block(i * NBUF + b, b)
        return carry

    lax.fori_loop(0, (ROWS_BASE + 1 + NBUF - 1) // NBUF, body, 0)

    for b in range(NBUF):
        pltpu.make_async_copy(rows[b], acc_sp.at[idxd_v.at[0]], ss[b]).wait()

    def drain(g, carry):
        pltpu.make_async_copy(ones_v, deg_sp.at[idxd_v.at[0]], semd).wait()
        return carry

    lax.fori_loop(0, nrows, drain, 0)
    plsc.subcore_barrier()
    pltpu.sync_copy(acc_sp.at[sl_acc], part_hbm.at[c].at[sl_acc])
    pltpu.sync_copy(deg_sp.at[sl_deg], indeg_hbm.at[c].at[sl_deg])


BLK = 1024


def _matmul_body(deg_ref, x_ref, w_ref, y_ref):
    deg = deg_ref[0, :] + deg_ref[1, :]
    norm = lax.rsqrt(jnp.maximum(deg, 1.0))
    y_ref[...] = jnp.dot(x_ref[...] * norm[:, None], w_ref[...],
                         preferred_element_type=jnp.float32)


_matmul = pl.pallas_call(
    _matmul_body,
    grid=(pl.cdiv(N_NODES, BLK),),
    in_specs=[
        pl.BlockSpec((2, BLK), lambda i: (0, i)),
        pl.BlockSpec((BLK, F_IN), lambda i: (i, 0)),
        pl.BlockSpec((F_IN, F_OUT), lambda i: (0, 0)),
    ],
    out_specs=pl.BlockSpec((BLK, F_OUT), lambda i: (i, 0)),
    out_shape=jax.ShapeDtypeStruct((N_NODES, F_OUT), jnp.float32),
)


FIN_BASE = N_NODES // NW
FIN_EXTRA = N_NODES % NW
FIN_SZ = 336


def _newton_rsqrt(d):
    i = lax.bitcast_convert_type(d, jnp.int32)
    r = lax.bitcast_convert_type(jnp.int32(0x5F3759DF) - (i >> 1), jnp.float32)
    for _ in range(3):
        r = r * (1.5 - 0.5 * d * r * r)
    return r


@functools.partial(
    pl.kernel,
    out_type=jax.ShapeDtypeStruct((N_NODES, F_OUT), jnp.float32),
    mesh=_mesh,
    compiler_params=_sc_params,
    scratch_types=[
        pltpu.VMEM((FIN_BASE + 1, F_OUT), jnp.float32),
        pltpu.VMEM((FIN_BASE + 1, F_OUT), jnp.float32),
        pltpu.VMEM((FIN_SZ,), jnp.float32),
        pltpu.VMEM((FIN_SZ,), jnp.float32),
        pltpu.VMEM((FIN_SZ,), jnp.float32),
    ],
)
def _finalize_kernel(part_hbm, indeg_hbm, out_hbm, a0, a1, d0, d1, nrm):
    c = lax.axis_index("c")
    s = lax.axis_index("s")
    wid = s * NC + c
    rbase = FIN_BASE * wid + jnp.minimum(wid, FIN_EXTRA)
    count = FIN_BASE + jnp.where(wid < FIN_EXTRA, 1, 0)
    al = (rbase // 8) * 8
    delta = rbase - al
    pltpu.sync_copy(part_hbm.at[0].at[pl.ds(rbase, FIN_BASE)],
                    a0.at[pl.ds(0, FIN_BASE)])
    pltpu.sync_copy(part_hbm.at[1].at[pl.ds(rbase, FIN_BASE)],
                    a1.at[pl.ds(0, FIN_BASE)])

    @pl.when(wid < FIN_EXTRA)
    def _():
        pltpu.sync_copy(part_hbm.at[0].at[pl.ds(rbase + FIN_BASE, 1)],
                        a0.at[pl.ds(FIN_BASE, 1)])
        pltpu.sync_copy(part_hbm.at[1].at[pl.ds(rbase + FIN_BASE, 1)],
                        a1.at[pl.ds(FIN_BASE, 1)])

    pltpu.sync_copy(indeg_hbm.at[0].at[pl.ds(al, FIN_SZ)], d0)
    pltpu.sync_copy(indeg_hbm.at[1].at[pl.ds(al, FIN_SZ)], d1)
    for k in range(FIN_SZ // LANES):
        kk = pl.ds(k * LANES, LANES)
        deg = jnp.maximum(d0[kk] + d1[kk], 1.0)
        nrm[kk] = _newton_rsqrt(deg)

    def scale_row(r, carry):
        v = nrm[pl.ds(delta + r, LANES)]
        nv = jnp.broadcast_to(v[0], (LANES,))
        for f in range(F_OUT // LANES):
            ff = pl.ds(f * LANES, LANES)
            a0[r, ff] = (a0[r, ff] + a1[r, ff]) * nv
        return carry

    lax.fori_loop(0, count, scale_row, 0)
    pltpu.sync_copy(a0.at[pl.ds(0, FIN_BASE)],
                    out_hbm.at[pl.ds(rbase, FIN_BASE)])

    @pl.when(wid < FIN_EXTRA)
    def _():
        pltpu.sync_copy(a0.at[pl.ds(FIN_BASE, 1)],
                        out_hbm.at[pl.ds(rbase + FIN_BASE, 1)])


@jax.jit
def kernel(x, edge_index, W):
    e3d = jnp.reshape(edge_index.astype(jnp.int32), (2, NROWS, CH))
    z1 = jnp.zeros((DEG_PER_SUB,), jnp.float32)
    zrow = jnp.zeros((ACC_PER_SUB, F_OUT), jnp.float32)

    out_deg = _out_degree_kernel(e3d, z1)
    y = _matmul(out_deg, x, W)
    part, in_deg = _aggregate_kernel(y, e3d, zrow, z1)
    return _finalize_kernel(part, in_deg)


# speedup vs baseline: 1.1707x; 1.0154x over previous
import functools

import jax
import jax.numpy as jnp
from jax import lax
from jax.experimental import pallas as pl
from jax.experimental.pallas import tpu as pltpu
from jax.experimental.pallas import tpu_sc as plsc
from jax.experimental import layout as jex_layout

N_NODES = 10000
N_EDGES = 320000
F_IN = 128
F_OUT = 64

NC = 2
NS = 16
LANES = 16
NW = NC * NS

CH = 128
NROWS = N_EDGES // CH
ROWS_BASE = NROWS // NW
ROWS_EXTRA = NROWS % NW
N_PAD = 10240
DEG_PER_SUB = N_PAD // NS
ACC_PER_SUB = N_NODES // NS

_mesh = plsc.VectorSubcoreMesh(core_axis_name="c", subcore_axis_name="s")
_sc_params = pltpu.CompilerParams(use_tc_tiling_on_sc=False)


def _tile_rows(wid):
    base = ROWS_BASE * wid + jnp.minimum(wid, ROWS_EXTRA)
    n = ROWS_BASE + jnp.where(wid < ROWS_EXTRA, 1, 0)
    return base, n


def _load_tile_rows(src3d, base, wid, idx_v):
    pltpu.sync_copy(src3d.at[pl.ds(base, ROWS_BASE)],
                    idx_v.at[pl.ds(0, ROWS_BASE)])

    @pl.when(wid < ROWS_EXTRA)
    def _():
        pltpu.sync_copy(src3d.at[pl.ds(base + ROWS_BASE, 1)],
                        idx_v.at[pl.ds(ROWS_BASE, 1)])


@functools.partial(
    pl.kernel,
    out_type=jax.ShapeDtypeStruct((NC, N_PAD), jnp.float32),
    mesh=_mesh,
    compiler_params=_sc_params,
    scratch_types=[
        pltpu.VMEM((ROWS_BASE + 1, CH), jnp.int32),
        pltpu.VMEM((CH,), jnp.float32),
        pltpu.VMEM_SHARED((N_PAD,), jnp.float32),
        pltpu.SemaphoreType.DMA,
    ],
)
def _out_degree_kernel(e3d_hbm, z1_hbm, deg_hbm, idx_v, ones_v, deg_sp, sem):
    c = lax.axis_index("c")
    s = lax.axis_index("s")
    wid = s * NC + c
    base, nrows = _tile_rows(wid)
    sl = pl.ds(s * DEG_PER_SUB, DEG_PER_SUB)
    pltpu.sync_copy(z1_hbm, deg_sp.at[sl])
    for i in range(CH // LANES):
        ones_v[pl.ds(i * LANES, LANES)] = jnp.ones((LANES,), jnp.float32)
    _load_tile_rows(e3d_hbm.at[0], base, wid, idx_v)
    plsc.subcore_barrier()

    def body(g, carry):
        pltpu.async_copy(ones_v, deg_sp.at[idx_v.at[g]], sem, add=True)
        return carry

    lax.fori_loop(0, nrows, body, 0)

    def drain(g, carry):
        pltpu.make_async_copy(ones_v, deg_sp.at[idx_v.at[0]], sem).wait()
        return carry

    lax.fori_loop(0, nrows, drain, 0)
    plsc.subcore_barrier()
    pltpu.sync_copy(deg_sp.at[sl], deg_hbm.at[c].at[sl])


@functools.partial(
    pl.kernel,
    out_type=(
        jax.ShapeDtypeStruct((NC, N_NODES, F_OUT), jnp.float32),
        jax.ShapeDtypeStruct((NC, N_PAD), jnp.float32),
    ),
    mesh=_mesh,
    compiler_params=_sc_params,
    scratch_types=[
        pltpu.VMEM((ROWS_BASE + 1, CH), jnp.int32),
        pltpu.VMEM((ROWS_BASE + 1, CH), jnp.int32),
        [pltpu.VMEM((CH, F_OUT), jnp.float32)] * 8,
        pltpu.VMEM((CH,), jnp.float32),
        pltpu.VMEM_SHARED((N_NODES, F_OUT), jnp.float32),
        pltpu.VMEM_SHARED((N_PAD,), jnp.float32),
        [pltpu.SemaphoreType.DMA] * 8,
        [pltpu.SemaphoreType.DMA] * 8,
        pltpu.SemaphoreType.DMA,
    ],
)
def _aggregate_kernel(y_hbm, e3d_hbm, zrow_hbm, z1_hbm,
                      part_hbm, indeg_hbm,
                      idxs_v, idxd_v, rows, ones_v,
                      acc_sp, deg_sp, sg, ss, semd):
    c = lax.axis_index("c")
    s = lax.axis_index("s")
    wid = s * NC + c
    base, nrows = _tile_rows(wid)
    sl_acc = pl.ds(s * ACC_PER_SUB, ACC_PER_SUB)
    sl_deg = pl.ds(s * DEG_PER_SUB, DEG_PER_SUB)
    pltpu.sync_copy(zrow_hbm, acc_sp.at[sl_acc])
    pltpu.sync_copy(z1_hbm, deg_sp.at[sl_deg])
    for i in range(CH // LANES):
        ones_v[pl.ds(i * LANES, LANES)] = jnp.ones((LANES,), jnp.float32)
    _load_tile_rows(e3d_hbm.at[0], base, wid, idxs_v)
    _load_tile_rows(e3d_hbm.at[1], base, wid, idxd_v)
    plsc.subcore_barrier()

    NBUF = 8
    for b in range(NBUF):
        pltpu.async_copy(y_hbm.at[idxs_v.at[b]], rows[b], sg[b])

    def block(g, b):
        @pl.when(g < nrows)
        def _():
            pltpu.make_async_copy(y_hbm.at[idxs_v.at[g]], rows[b], sg[b]).wait()
            pltpu.async_copy(rows[b], acc_sp.at[idxd_v.at[g]], ss[b], add=True)
            pltpu.async_copy(ones_v, deg_sp.at[idxd_v.at[g]], semd, add=True)
            j = g - 4
            jb = (b - 4) % NBUF

            @pl.when(jnp.logical_and(j >= 0, j + NBUF < nrows))
            def _():
                pltpu.make_async_copy(rows[jb], acc_sp.at[idxd_v.at[0]],
                                      ss[jb]).wait()
                pltpu.async_copy(y_hbm.at[idxs_v.at[j + NBUF]], rows[jb],
                                 sg[jb])

    def body(i, carry):
        for b in range(NBUF):
            block(i * NBUF + b, b)
        return carry

    lax.fori_loop(0, (ROWS_BASE + 1 + NBUF - 1) // NBUF, body, 0)

    for b in range(NBUF):
        pltpu.make_async_copy(rows[b], acc_sp.at[idxd_v.at[0]], ss[b]).wait()

    def drain(g, carry):
        pltpu.make_async_copy(ones_v, deg_sp.at[idxd_v.at[0]], semd).wait()
        return carry

    lax.fori_loop(0, nrows, drain, 0)
    plsc.subcore_barrier()
    pltpu.sync_copy(acc_sp.at[sl_acc], part_hbm.at[c].at[sl_acc])
    pltpu.sync_copy(deg_sp.at[sl_deg], indeg_hbm.at[c].at[sl_deg])


BLK = 2048


def _matmul_body(deg_ref, x_ref, w_ref, y_ref):
    deg = deg_ref[0, :] + deg_ref[1, :]
    norm = lax.rsqrt(jnp.maximum(deg, 1.0))
    y_ref[...] = jnp.dot(x_ref[...] * norm[:, None], w_ref[...],
                         preferred_element_type=jnp.float32)


_matmul = pl.pallas_call(
    _matmul_body,
    grid=(pl.cdiv(N_NODES, BLK),),
    in_specs=[
        pl.BlockSpec((2, BLK), lambda i: (0, i)),
        pl.BlockSpec((BLK, F_IN), lambda i: (i, 0)),
        pl.BlockSpec((F_IN, F_OUT), lambda i: (0, 0)),
    ],
    out_specs=pl.BlockSpec((BLK, F_OUT), lambda i: (i, 0)),
    out_shape=jax.ShapeDtypeStruct((N_NODES, F_OUT), jnp.float32),
)


FIN_BASE = N_NODES // NW
FIN_EXTRA = N_NODES % NW
FIN_SZ = 336


def _newton_rsqrt(d):
    i = lax.bitcast_convert_type(d, jnp.int32)
    r = lax.bitcast_convert_type(jnp.int32(0x5F3759DF) - (i >> 1), jnp.float32)
    for _ in range(3):
        r = r * (1.5 - 0.5 * d * r * r)
    return r


@functools.partial(
    pl.kernel,
    out_type=jax.ShapeDtypeStruct((N_NODES, F_OUT), jnp.float32),
    mesh=_mesh,
    compiler_params=_sc_params,
    scratch_types=[
        pltpu.VMEM((FIN_BASE + 1, F_OUT), jnp.float32),
        pltpu.VMEM((FIN_BASE + 1, F_OUT), jnp.float32),
        pltpu.VMEM((FIN_SZ,), jnp.float32),
        pltpu.VMEM((FIN_SZ,), jnp.float32),
        pltpu.VMEM((FIN_SZ,), jnp.float32),
    ],
)
def _finalize_kernel(part_hbm, indeg_hbm, out_hbm, a0, a1, d0, d1, nrm):
    c = lax.axis_index("c")
    s = lax.axis_index("s")
    wid = s * NC + c
    rbase = FIN_BASE * wid + jnp.minimum(wid, FIN_EXTRA)
    count = FIN_BASE + jnp.where(wid < FIN_EXTRA, 1, 0)
    al = (rbase // 8) * 8
    delta = rbase - al
    pltpu.sync_copy(part_hbm.at[0].at[pl.ds(rbase, FIN_BASE)],
                    a0.at[pl.ds(0, FIN_BASE)])
    pltpu.sync_copy(part_hbm.at[1].at[pl.ds(rbase, FIN_BASE)],
                    a1.at[pl.ds(0, FIN_BASE)])

    @pl.when(wid < FIN_EXTRA)
    def _():
        pltpu.sync_copy(part_hbm.at[0].at[pl.ds(rbase + FIN_BASE, 1)],
                        a0.at[pl.ds(FIN_BASE, 1)])
        pltpu.sync_copy(part_hbm.at[1].at[pl.ds(rbase + FIN_BASE, 1)],
                        a1.at[pl.ds(FIN_BASE, 1)])

    pltpu.sync_copy(indeg_hbm.at[0].at[pl.ds(al, FIN_SZ)], d0)
    pltpu.sync_copy(indeg_hbm.at[1].at[pl.ds(al, FIN_SZ)], d1)
    for k in range(FIN_SZ // LANES):
        kk = pl.ds(k * LANES, LANES)
        deg = jnp.maximum(d0[kk] + d1[kk], 1.0)
        nrm[kk] = _newton_rsqrt(deg)

    def scale_row(r, carry):
        v = nrm[pl.ds(delta + r, LANES)]
        nv = jnp.broadcast_to(v[0], (LANES,))
        for f in range(F_OUT // LANES):
            ff = pl.ds(f * LANES, LANES)
            a0[r, ff] = (a0[r, ff] + a1[r, ff]) * nv
        return carry

    lax.fori_loop(0, count, scale_row, 0)
    pltpu.sync_copy(a0.at[pl.ds(0, FIN_BASE)],
                    out_hbm.at[pl.ds(rbase, FIN_BASE)])

    @pl.when(wid < FIN_EXTRA)
    def _():
        pltpu.sync_copy(a0.at[pl.ds(FIN_BASE, 1)],
                        out_hbm.at[pl.ds(rbase + FIN_BASE, 1)])


@jax.jit
def kernel(x, edge_index, W):
    e3d = jnp.reshape(edge_index.astype(jnp.int32), (2, NROWS, CH))
    z1 = jnp.zeros((DEG_PER_SUB,), jnp.float32)
    zrow = jnp.zeros((ACC_PER_SUB, F_OUT), jnp.float32)

    out_deg = _out_degree_kernel(e3d, z1)
    y = _matmul(out_deg, x, W)
    part, in_deg = _aggregate_kernel(y, e3d, zrow, z1)
    out = _finalize_kernel(part, in_deg)
    return jex_layout.with_layout_constraint(
        out, jex_layout.Layout((0, 1), ((8,),)))
